# baseline (speedup 1.0000x reference)
import functools
import math

import jax
import jax.numpy as jnp
import numpy as np
from jax import lax
from jax.experimental import pallas as pl
from jax.experimental.pallas import tpu as pltpu
from jax.experimental.pallas import tpu_sc as plsc

VOCAB = 100000
DIM = 1024
SEQ = 2048
BATCH = 4
TOKENS = BATCH * SEQ
LN_EPS = 1e-12
EPS_SCALED = LN_EPS / 1024.0

NUM_WORKERS = 32
NSPLIT = 1
CTOK = TOKENS // NSPLIT
CBATCH = BATCH // NSPLIT
TOK_PER_W = CTOK // NUM_WORKERS
CH = 32
NCH = TOK_PER_W // CH
NBUF = 3
TB = 2048
SBLK = SEQ // TB


@functools.lru_cache(maxsize=1)
def _pe_div32() -> np.ndarray:
    position = np.arange(0, SEQ, dtype=np.float64)[:, None]
    div_term = np.exp(np.arange(0, DIM, 2, dtype=np.float64) * -(math.log(10000.0) / DIM))
    pe = np.zeros((SEQ, DIM), dtype=np.float32)
    pe[:, 0::2] = np.sin(position * div_term).astype(np.float32)
    pe[:, 1::2] = np.cos(position * div_term).astype(np.float32)
    return pe / np.float32(math.sqrt(DIM))


def _sc_gather_body(ids_hbm, word_hbm, out_hbm, idx_v, wbuf0, wbuf1, wbuf2,
                    gs0, gs1, gs2, ws0, ws1, ws2):
    wid = lax.axis_index("s") * 2 + lax.axis_index("c")
    base = wid * TOK_PER_W
    pltpu.sync_copy(ids_hbm.at[pl.ds(base, TOK_PER_W)], idx_v)

    bufs = (wbuf0, wbuf1, wbuf2)
    gsems = (gs0, gs1, gs2)
    wsems = (ws0, ws1, ws2)

    def start_gather(c):
        p = c % NBUF
        return pltpu.async_copy(
            word_hbm.at[idx_v.at[pl.ds(c * CH, CH)]], bufs[p], gsems[p])

    gets = [None] * NCH
    writes = [None] * NCH
    for c in range(NBUF):
        gets[c] = start_gather(c)
    for c in range(NCH):
        p = c % NBUF
        gets[c].wait()
        writes[c] = pltpu.async_copy(
            bufs[p], out_hbm.at[pl.ds(base + c * CH, CH)], wsems[p])
        if c + NBUF < NCH:
            writes[c].wait()
            gets[c + NBUF] = start_gather(c + NBUF)
    for c in range(NCH - NBUF, NCH):
        writes[c].wait()


def _sc_gather(ids_flat, word_table):
    mesh = plsc.VectorSubcoreMesh(core_axis_name="c", subcore_axis_name="s")
    run = functools.partial(
        pl.kernel,
        mesh=mesh,
        out_type=jax.ShapeDtypeStruct((CTOK, DIM), jnp.float32),
        scratch_types=[
            pltpu.VMEM((TOK_PER_W,), jnp.int32),
            pltpu.VMEM((CH, DIM), jnp.float32),
            pltpu.VMEM((CH, DIM), jnp.float32),
            pltpu.VMEM((CH, DIM), jnp.float32),
            pltpu.SemaphoreType.DMA,
            pltpu.SemaphoreType.DMA,
            pltpu.SemaphoreType.DMA,
            pltpu.SemaphoreType.DMA,
            pltpu.SemaphoreType.DMA,
            pltpu.SemaphoreType.DMA,
        ],
    )(_sc_gather_body)
    return run(ids_flat, word_table)


def _tc_ln_body(w_ref, tt_ref, pe_ref, type_ref, g_ref, b_ref, *rest):
    o_ref = rest[-1]
    w = w_ref[...]
    pe = pe_ref[...]
    tt = tt_ref[...].astype(jnp.float32)
    t0 = type_ref[0:1, :]
    t1 = type_ref[1:2, :]
    x = w + pe + t0 + tt * (t1 - t0)
    m = jnp.mean(x, axis=1, keepdims=True)
    var = jnp.mean(x * x, axis=1, keepdims=True) - m * m
    r = lax.rsqrt(var + EPS_SCALED)
    o_ref[...] = (x - m) * (r * g_ref[...]) + b_ref[...]


def _tc_ln(gathered, tts_col, pe32, type_table, gamma2d, beta2d, q, prev):
    in_specs = [
        pl.BlockSpec((TB, DIM), lambda s, b: (b * SBLK + s, 0)),
        pl.BlockSpec((TB, 1), lambda s, b: (b * SBLK + s, 0)),
        pl.BlockSpec((TB, DIM), lambda s, b: (s, 0)),
        pl.BlockSpec((2, DIM), lambda s, b: (0, 0)),
        pl.BlockSpec((1, DIM), lambda s, b: (0, 0)),
        pl.BlockSpec((1, DIM), lambda s, b: (0, 0)),
    ]
    args = [gathered, tts_col, pe32, type_table, gamma2d, beta2d]
    aliases = {}
    if prev is not None:
        in_specs.append(pl.BlockSpec(memory_space=pl.ANY))
        args.append(prev)
        aliases = {6: 0}
    row0 = q * (CTOK // TB)
    return pl.pallas_call(
        _tc_ln_body,
        grid=(SBLK, CBATCH),
        in_specs=in_specs,
        out_specs=pl.BlockSpec((TB, DIM), lambda s, b: (row0 + b * SBLK + s, 0)),
        out_shape=jax.ShapeDtypeStruct((TOKENS, DIM), jnp.float32),
        input_output_aliases=aliases,
        compiler_params=pltpu.CompilerParams(
            dimension_semantics=("arbitrary", "arbitrary")),
    )(*args)


def kernel(input_ids, token_type_ids, word_table, type_table, ln_gamma, ln_beta):
    ids_flat = input_ids.reshape(TOKENS).astype(jnp.int32)
    tts_col = token_type_ids.reshape(TOKENS, 1).astype(jnp.int32)
    pe32 = jnp.asarray(_pe_div32())

    gamma2d = ln_gamma.reshape(1, DIM)
    beta2d = ln_beta.reshape(1, DIM)
    out = None
    for q in range(NSPLIT):
        lo = q * CTOK
        gathered = _sc_gather(ids_flat[lo:lo + CTOK], word_table)
        out = _tc_ln(gathered, tts_col[lo:lo + CTOK], pe32, type_table,
                     gamma2d, beta2d, q, out)
    return out.reshape(BATCH, SEQ, DIM)


# speedup vs baseline: 1.0175x; 1.0175x over previous
import functools
import math

import jax
import jax.numpy as jnp
import numpy as np
from jax import lax
from jax.experimental import pallas as pl
from jax.experimental.pallas import tpu as pltpu
from jax.experimental.pallas import tpu_sc as plsc

VOCAB = 100000
DIM = 1024
SEQ = 2048
BATCH = 4
TOKENS = BATCH * SEQ
LN_EPS = 1e-12
EPS_SCALED = LN_EPS / 1024.0

NUM_WORKERS = 32
NSPLIT = 1
CTOK = TOKENS // NSPLIT
CBATCH = BATCH // NSPLIT
TOK_PER_W = CTOK // NUM_WORKERS
CH = 32
NCH = TOK_PER_W // CH
NBUF = 3
TB = 2048
SBLK = SEQ // TB


@functools.lru_cache(maxsize=1)
def _pe_div32() -> np.ndarray:
    position = np.arange(0, SEQ, dtype=np.float64)[:, None]
    div_term = np.exp(np.arange(0, DIM, 2, dtype=np.float64) * -(math.log(10000.0) / DIM))
    pe = np.zeros((SEQ, DIM), dtype=np.float32)
    pe[:, 0::2] = np.sin(position * div_term).astype(np.float32)
    pe[:, 1::2] = np.cos(position * div_term).astype(np.float32)
    return pe / np.float32(math.sqrt(DIM))


def _sc_gather_body(ids_hbm, word_hbm, out_hbm, idx_v, wbuf0, wbuf1, wbuf2,
                    gs0, gs1, gs2, ws0, ws1, ws2):
    wid = lax.axis_index("s") * 2 + lax.axis_index("c")
    base = wid * TOK_PER_W
    pltpu.sync_copy(ids_hbm.at[pl.ds(base, TOK_PER_W)], idx_v)

    bufs = (wbuf0, wbuf1, wbuf2)
    gsems = (gs0, gs1, gs2)
    wsems = (ws0, ws1, ws2)

    def start_gather(c):
        p = c % NBUF
        return pltpu.async_copy(
            word_hbm.at[idx_v.at[pl.ds(c * CH, CH)]], bufs[p], gsems[p])

    gets = [None] * NCH
    writes = [None] * NCH
    for c in range(NBUF):
        gets[c] = start_gather(c)
    for c in range(NCH):
        p = c % NBUF
        gets[c].wait()
        writes[c] = pltpu.async_copy(
            bufs[p], out_hbm.at[pl.ds(base + c * CH, CH)], wsems[p])
        if c + NBUF < NCH:
            writes[c].wait()
            gets[c + NBUF] = start_gather(c + NBUF)
    for c in range(NCH - NBUF, NCH):
        writes[c].wait()


def _sc_gather(ids_flat, word_table):
    mesh = plsc.VectorSubcoreMesh(core_axis_name="c", subcore_axis_name="s")
    run = functools.partial(
        pl.kernel,
        mesh=mesh,
        out_type=jax.ShapeDtypeStruct((CTOK, DIM), jnp.float32),
        scratch_types=[
            pltpu.VMEM((TOK_PER_W,), jnp.int32),
            pltpu.VMEM((CH, DIM), jnp.float32),
            pltpu.VMEM((CH, DIM), jnp.float32),
            pltpu.VMEM((CH, DIM), jnp.float32),
            pltpu.SemaphoreType.DMA,
            pltpu.SemaphoreType.DMA,
            pltpu.SemaphoreType.DMA,
            pltpu.SemaphoreType.DMA,
            pltpu.SemaphoreType.DMA,
            pltpu.SemaphoreType.DMA,
        ],
    )(_sc_gather_body)
    return run(ids_flat, word_table)


def _tc_ln_body(w_ref, tt_ref, pe_ref, type_ref, g_ref, b_ref, *rest):
    o_ref = rest[-1]
    w = w_ref[...]
    pe = pe_ref[...]
    tt = tt_ref[...].astype(jnp.float32).reshape(TB, 1)
    t0 = type_ref[0:1, :]
    t1 = type_ref[1:2, :]
    x = w + pe + t0 + tt * (t1 - t0)
    m = jnp.mean(x, axis=1, keepdims=True)
    var = jnp.mean(x * x, axis=1, keepdims=True) - m * m
    r = lax.rsqrt(var + EPS_SCALED)
    o_ref[...] = (x - m) * (r * g_ref[...]) + b_ref[...]


def _tc_ln(gathered, tts_rows, pe32, type_table, gamma2d, beta2d, q, prev):
    in_specs = [
        pl.BlockSpec((TB, DIM), lambda s, b: (b * SBLK + s, 0)),
        pl.BlockSpec((1, 1, TB), lambda s, b: (q * CBATCH + b, 0, s)),
        pl.BlockSpec((TB, DIM), lambda s, b: (s, 0)),
        pl.BlockSpec((2, DIM), lambda s, b: (0, 0)),
        pl.BlockSpec((1, DIM), lambda s, b: (0, 0)),
        pl.BlockSpec((1, DIM), lambda s, b: (0, 0)),
    ]
    args = [gathered, tts_rows, pe32, type_table, gamma2d, beta2d]
    aliases = {}
    if prev is not None:
        in_specs.append(pl.BlockSpec(memory_space=pl.ANY))
        args.append(prev)
        aliases = {6: 0}
    row0 = q * (CTOK // TB)
    return pl.pallas_call(
        _tc_ln_body,
        grid=(SBLK, CBATCH),
        in_specs=in_specs,
        out_specs=pl.BlockSpec((TB, DIM), lambda s, b: (row0 + b * SBLK + s, 0)),
        out_shape=jax.ShapeDtypeStruct((TOKENS, DIM), jnp.float32),
        input_output_aliases=aliases,
        compiler_params=pltpu.CompilerParams(
            dimension_semantics=("arbitrary", "arbitrary")),
    )(*args)


def kernel(input_ids, token_type_ids, word_table, type_table, ln_gamma, ln_beta):
    ids_flat = input_ids.reshape(TOKENS).astype(jnp.int32)
    tts_rows = token_type_ids.astype(jnp.int32).reshape(BATCH, 1, SEQ)
    pe32 = jnp.asarray(_pe_div32())

    gamma2d = ln_gamma.reshape(1, DIM)
    beta2d = ln_beta.reshape(1, DIM)
    out = None
    for q in range(NSPLIT):
        lo = q * CTOK
        gathered = _sc_gather(ids_flat[lo:lo + CTOK], word_table)
        out = _tc_ln(gathered, tts_rows, pe32, type_table,
                     gamma2d, beta2d, q, out)
    return out.reshape(BATCH, SEQ, DIM)
